# Initial kernel scaffold; baseline (speedup 1.0000x reference)
import jax
import jax.numpy as jnp
from jax.experimental import pallas as pl


def kernel(x, R_diagonal, edge_index, edge_type):
    raise NotImplementedError("write your pallas kernel here")


# speedup vs baseline: 1.8823x; 1.8823x over previous
import functools

import jax
import jax.numpy as jnp
from jax.experimental import pallas as pl
from jax.experimental.pallas import tpu as pltpu

_MIB = 1024 * 1024


def _round_up(a: int, b: int) -> int:
    return (a + b - 1) // b * b


def _norm_kernel(x_ref, out_ref):
    xb = x_ref[...]
    ss = jnp.sum(xb * xb, axis=1, keepdims=True)
    inv = jax.lax.rsqrt(jnp.maximum(ss, jnp.float32(1e-24)))
    out_ref[...] = (xb * inv).astype(jnp.bfloat16)


def _score_kernel(rT_ref, et_ref, s_ref, o_ref, out_ref, *, n_rel_pad: int):
    s = s_ref[...].astype(jnp.float32)
    o = o_ref[...].astype(jnp.float32)
    t = s * o
    et = et_ref[0]
    krel = jax.lax.broadcasted_iota(jnp.int32, (n_rel_pad, t.shape[1]), 0)
    onehot = (krel == et).astype(jnp.bfloat16)
    r = jnp.dot(rT_ref[...], onehot, preferred_element_type=jnp.float32)
    out_ref[...] = jnp.sum(t * r, axis=0, keepdims=True)


def _distmult(x, R_diagonal, edge_index, edge_type, *,
              tile_n: int = 2000, tile_e: int = 2048):
    E = int(edge_index.shape[1])
    C = int(x.shape[1])
    N = int(x.shape[0])
    n_rel = int(R_diagonal.shape[0])
    n_rel_pad = _round_up(n_rel, 256)

    tile_n = min(tile_n, _round_up(N, 8))
    N_pad = _round_up(N, tile_n)
    x_p = jnp.pad(x, ((0, N_pad - N), (0, 0))) if N_pad != N else x
    xn = pl.pallas_call(
        _norm_kernel,
        out_shape=jax.ShapeDtypeStruct((N_pad, C), jnp.bfloat16),
        grid=(N_pad // tile_n,),
        in_specs=[pl.BlockSpec((tile_n, C), lambda i: (i, 0))],
        out_specs=pl.BlockSpec((tile_n, C), lambda i: (i, 0)),
        compiler_params=pltpu.CompilerParams(
            dimension_semantics=("parallel",)),
    )(x_p)

    xnT = xn[:N].T
    sT = jnp.take(xnT, edge_index[0, :], axis=1)
    oT = jnp.take(xnT, edge_index[1, :], axis=1)

    tile_e = min(tile_e, _round_up(max(E, 1), 128))
    E_pad = _round_up(max(E, 1), tile_e)
    if E_pad != E:
        sT = jnp.pad(sT, ((0, 0), (0, E_pad - E)))
        oT = jnp.pad(oT, ((0, 0), (0, E_pad - E)))
        et = jnp.pad(edge_type.astype(jnp.int32), (0, E_pad - E))
    else:
        et = edge_type.astype(jnp.int32)
    n_tiles = E_pad // tile_e
    et3 = et.reshape(n_tiles, 1, tile_e)

    rT = jnp.pad(R_diagonal, ((0, n_rel_pad - n_rel), (0, 0))
                 ).T.astype(jnp.bfloat16)

    score = pl.pallas_call(
        functools.partial(_score_kernel, n_rel_pad=n_rel_pad),
        out_shape=jax.ShapeDtypeStruct((1, E_pad), jnp.float32),
        grid=(n_tiles,),
        in_specs=[
            pl.BlockSpec((C, n_rel_pad), lambda i: (0, 0)),
            pl.BlockSpec((1, 1, tile_e), lambda i: (i, 0, 0)),
            pl.BlockSpec((C, tile_e), lambda i: (0, i)),
            pl.BlockSpec((C, tile_e), lambda i: (0, i)),
        ],
        out_specs=pl.BlockSpec((1, tile_e), lambda i: (0, i)),
        compiler_params=pltpu.CompilerParams(
            dimension_semantics=("parallel",),
            vmem_limit_bytes=48 * _MIB),
    )(rT, et3, sT, oT)

    return score[0, :E]


def kernel(x, R_diagonal, edge_index, edge_type):
    return _distmult(x, R_diagonal, edge_index, edge_type)


# speedup vs baseline: 1.9895x; 1.0569x over previous
import functools

import jax
import jax.numpy as jnp
from jax.experimental import pallas as pl
from jax.experimental.pallas import tpu as pltpu

_MIB = 1024 * 1024


def _round_up(a: int, b: int) -> int:
    return (a + b - 1) // b * b


def _norm_kernel(x_ref, out_ref):
    xb = x_ref[...]
    ss = jnp.sum(xb * xb, axis=1, keepdims=True)
    inv = jax.lax.rsqrt(jnp.maximum(ss, jnp.float32(1e-24)))
    out_ref[...] = (xb * inv).astype(jnp.bfloat16)


def _score_kernel(r_ref, et_ref, s_ref, o_ref, out_ref, *, n_rel_pad: int):
    s = s_ref[...].astype(jnp.float32)
    o = o_ref[...].astype(jnp.float32)
    t = (s * o).astype(jnp.bfloat16)
    q = jax.lax.dot_general(r_ref[...], t, (((1,), (1,)), ((), ())),
                            preferred_element_type=jnp.float32)
    et = et_ref[0]
    krel = jax.lax.broadcasted_iota(jnp.int32, q.shape, 0)
    sel = jnp.where(krel == et, q, jnp.float32(0.0))
    out_ref[...] = jnp.sum(sel, axis=0, keepdims=True)


def _distmult(x, R_diagonal, edge_index, edge_type, *,
              tile_n: int = 2000, tile_e: int = 2048):
    E = int(edge_index.shape[1])
    C = int(x.shape[1])
    N = int(x.shape[0])
    n_rel = int(R_diagonal.shape[0])
    n_rel_pad = _round_up(n_rel, 256)

    tile_n = min(tile_n, _round_up(N, 8))
    N_pad = _round_up(N, tile_n)
    x_p = jnp.pad(x, ((0, N_pad - N), (0, 0))) if N_pad != N else x
    xn = pl.pallas_call(
        _norm_kernel,
        out_shape=jax.ShapeDtypeStruct((N_pad, C), jnp.bfloat16),
        grid=(N_pad // tile_n,),
        in_specs=[pl.BlockSpec((tile_n, C), lambda i: (i, 0))],
        out_specs=pl.BlockSpec((tile_n, C), lambda i: (i, 0)),
        compiler_params=pltpu.CompilerParams(
            dimension_semantics=("parallel",)),
    )(x_p)

    s = jnp.take(xn[:N], edge_index[0, :], axis=0)
    o = jnp.take(xn[:N], edge_index[1, :], axis=0)

    tile_e = min(tile_e, _round_up(max(E, 1), 128))
    E_pad = _round_up(max(E, 1), tile_e)
    if E_pad != E:
        s = jnp.pad(s, ((0, E_pad - E), (0, 0)))
        o = jnp.pad(o, ((0, E_pad - E), (0, 0)))
        et = jnp.pad(edge_type.astype(jnp.int32), (0, E_pad - E))
    else:
        et = edge_type.astype(jnp.int32)
    n_tiles = E_pad // tile_e
    et3 = et.reshape(n_tiles, 1, tile_e)

    r_pad = jnp.pad(R_diagonal, ((0, n_rel_pad - n_rel), (0, 0))
                    ).astype(jnp.bfloat16)

    score = pl.pallas_call(
        functools.partial(_score_kernel, n_rel_pad=n_rel_pad),
        out_shape=jax.ShapeDtypeStruct((1, E_pad), jnp.float32),
        grid=(n_tiles,),
        in_specs=[
            pl.BlockSpec((n_rel_pad, C), lambda i: (0, 0)),
            pl.BlockSpec((1, 1, tile_e), lambda i: (i, 0, 0)),
            pl.BlockSpec((tile_e, C), lambda i: (i, 0)),
            pl.BlockSpec((tile_e, C), lambda i: (i, 0)),
        ],
        out_specs=pl.BlockSpec((1, tile_e), lambda i: (0, i)),
        compiler_params=pltpu.CompilerParams(
            dimension_semantics=("parallel",),
            vmem_limit_bytes=48 * _MIB),
    )(r_pad, et3, s, o)

    return score[0, :E]


def kernel(x, R_diagonal, edge_index, edge_type):
    return _distmult(x, R_diagonal, edge_index, edge_type)


# speedup vs baseline: 5.0321x; 2.5293x over previous
import functools

import jax
import jax.numpy as jnp
from jax.experimental import pallas as pl
from jax.experimental.pallas import tpu as pltpu

_MIB = 1024 * 1024


def _round_up(a: int, b: int) -> int:
    return (a + b - 1) // b * b


def _norm_kernel(x_ref, out_ref):
    xb = x_ref[...]
    ss = jnp.sum(xb * xb, axis=1, keepdims=True)
    inv = jax.lax.rsqrt(jnp.maximum(ss, jnp.float32(1e-24)))
    out_ref[...] = xb * inv


def _score_kernel(r_ref, et_ref, sidx_ref, didx_ref, table_ref, out_ref,
                  s_tile, o_tile, *, mc: int):
    for mi in range(mc):
        si = sidx_ref[0, 0, mi]
        di = didx_ref[0, 0, mi]
        s_tile[mi, :] = table_ref[si, 0]
        o_tile[mi, :] = table_ref[di, 0]
    t = (s_tile[...] * o_tile[...]).astype(jnp.bfloat16)
    q = jax.lax.dot_general(r_ref[...], t, (((1,), (1,)), ((), ())),
                            preferred_element_type=jnp.float32)
    et = et_ref[0]
    krel = jax.lax.broadcasted_iota(jnp.int32, q.shape, 0)
    sel = jnp.where(krel == et, q, jnp.float32(0.0))
    out_ref[...] = jnp.sum(sel, axis=0, keepdims=True)


def _distmult(x, R_diagonal, edge_index, edge_type, *,
              tile_n: int = 2000, mc: int = 512):
    E = int(edge_index.shape[1])
    C = int(x.shape[1])
    N = int(x.shape[0])
    n_rel = int(R_diagonal.shape[0])
    n_rel_pad = _round_up(n_rel, 256)

    tile_n = min(tile_n, _round_up(N, 8))
    N_pad = _round_up(N, tile_n)
    x_p = jnp.pad(x, ((0, N_pad - N), (0, 0))) if N_pad != N else x
    xn = pl.pallas_call(
        _norm_kernel,
        out_shape=jax.ShapeDtypeStruct((N_pad, C), jnp.float32),
        grid=(N_pad // tile_n,),
        in_specs=[pl.BlockSpec((tile_n, C), lambda i: (i, 0))],
        out_specs=pl.BlockSpec((tile_n, C), lambda i: (i, 0)),
        compiler_params=pltpu.CompilerParams(
            dimension_semantics=("parallel",)),
    )(x_p)
    xn3 = xn.reshape(N_pad, 1, C)

    mc = min(mc, _round_up(max(E, 1), 128))
    E_pad = _round_up(max(E, 1), mc)
    src = edge_index[0, :].astype(jnp.int32)
    dst = edge_index[1, :].astype(jnp.int32)
    et = edge_type.astype(jnp.int32)
    if E_pad != E:
        src = jnp.pad(src, (0, E_pad - E))
        dst = jnp.pad(dst, (0, E_pad - E))
        et = jnp.pad(et, (0, E_pad - E))
    n_steps = E_pad // mc
    src3 = src.reshape(n_steps, 1, mc)
    dst3 = dst.reshape(n_steps, 1, mc)
    et3 = et.reshape(n_steps, 1, mc)

    r_pad = jnp.pad(R_diagonal, ((0, n_rel_pad - n_rel), (0, 0))
                    ).astype(jnp.bfloat16)

    score = pl.pallas_call(
        functools.partial(_score_kernel, mc=mc),
        out_shape=jax.ShapeDtypeStruct((1, E_pad), jnp.float32),
        grid=(n_steps,),
        in_specs=[
            pl.BlockSpec((n_rel_pad, C), lambda i: (0, 0)),
            pl.BlockSpec((1, 1, mc), lambda i: (i, 0, 0)),
            pl.BlockSpec((1, 1, mc), lambda i: (i, 0, 0),
                         memory_space=pltpu.SMEM),
            pl.BlockSpec((1, 1, mc), lambda i: (i, 0, 0),
                         memory_space=pltpu.SMEM),
            pl.BlockSpec((N_pad, 1, C), lambda i: (0, 0, 0)),
        ],
        out_specs=pl.BlockSpec((1, mc), lambda i: (0, i)),
        scratch_shapes=[
            pltpu.VMEM((mc, C), jnp.float32),
            pltpu.VMEM((mc, C), jnp.float32),
        ],
        compiler_params=pltpu.CompilerParams(
            dimension_semantics=("parallel",),
            vmem_limit_bytes=56 * _MIB),
    )(r_pad, et3, src3, dst3, xn3)

    return score[0, :E]


def kernel(x, R_diagonal, edge_index, edge_type):
    return _distmult(x, R_diagonal, edge_index, edge_type)


# speedup vs baseline: 5.6359x; 1.1200x over previous
import functools

import jax
import jax.numpy as jnp
from jax.experimental import pallas as pl
from jax.experimental.pallas import tpu as pltpu

_MIB = 1024 * 1024


def _round_up(a: int, b: int) -> int:
    return (a + b - 1) // b * b


def _norm_kernel(x_ref, out_ref):
    xb = x_ref[...]
    ss = jnp.sum(xb * xb, axis=1, keepdims=True)
    inv = jax.lax.rsqrt(jnp.maximum(ss, jnp.float32(1e-24)))
    out_ref[...] = xb * inv


def _score_kernel(r_ref, et_ref, sidx_ref, didx_ref, table_ref, out_ref,
                  s_tile, o_tile, *, mc: int, sub: int):
    for c0 in range(0, mc, sub):
        for mi in range(c0, c0 + sub):
            si = sidx_ref[0, 0, mi]
            di = didx_ref[0, 0, mi]
            s_tile[mi, :] = table_ref[si, 0]
            o_tile[mi, :] = table_ref[di, 0]
        chunk = pl.ds(c0, sub)
        t = (s_tile[chunk, :] * o_tile[chunk, :]).astype(jnp.bfloat16)
        q = jax.lax.dot_general(r_ref[...], t, (((1,), (1,)), ((), ())),
                                preferred_element_type=jnp.float32)
        et = et_ref[0, :, chunk]
        krel = jax.lax.broadcasted_iota(jnp.int32, q.shape, 0)
        sel = jnp.where(krel == et, q, jnp.float32(0.0))
        out_ref[0, chunk] = jnp.sum(sel, axis=0)


def _distmult(x, R_diagonal, edge_index, edge_type, *,
              tile_n: int = 2000, mc: int = 1024, sub: int = 256):
    E = int(edge_index.shape[1])
    C = int(x.shape[1])
    N = int(x.shape[0])
    n_rel = int(R_diagonal.shape[0])
    n_rel_pad = _round_up(n_rel, 256)

    tile_n = min(tile_n, _round_up(N, 8))
    N_pad = _round_up(N, tile_n)
    x_p = jnp.pad(x, ((0, N_pad - N), (0, 0))) if N_pad != N else x
    xn = pl.pallas_call(
        _norm_kernel,
        out_shape=jax.ShapeDtypeStruct((N_pad, C), jnp.float32),
        grid=(N_pad // tile_n,),
        in_specs=[pl.BlockSpec((tile_n, C), lambda i: (i, 0))],
        out_specs=pl.BlockSpec((tile_n, C), lambda i: (i, 0)),
        compiler_params=pltpu.CompilerParams(
            dimension_semantics=("parallel",)),
    )(x_p)
    xn3 = xn.reshape(N_pad, 1, C)

    mc = min(mc, _round_up(max(E, 1), 128))
    E_pad = _round_up(max(E, 1), mc)
    src = edge_index[0, :].astype(jnp.int32)
    dst = edge_index[1, :].astype(jnp.int32)
    et = edge_type.astype(jnp.int32)
    if E_pad != E:
        src = jnp.pad(src, (0, E_pad - E))
        dst = jnp.pad(dst, (0, E_pad - E))
        et = jnp.pad(et, (0, E_pad - E))
    n_steps = E_pad // mc
    src3 = src.reshape(n_steps, 1, mc)
    dst3 = dst.reshape(n_steps, 1, mc)
    et3 = et.reshape(n_steps, 1, mc)

    r_pad = jnp.pad(R_diagonal, ((0, n_rel_pad - n_rel), (0, 0))
                    ).astype(jnp.bfloat16)

    score = pl.pallas_call(
        functools.partial(_score_kernel, mc=mc, sub=min(sub, mc)),
        out_shape=jax.ShapeDtypeStruct((1, E_pad), jnp.float32),
        grid=(n_steps,),
        in_specs=[
            pl.BlockSpec((n_rel_pad, C), lambda i: (0, 0)),
            pl.BlockSpec((1, 1, mc), lambda i: (i, 0, 0)),
            pl.BlockSpec((1, 1, mc), lambda i: (i, 0, 0),
                         memory_space=pltpu.SMEM),
            pl.BlockSpec((1, 1, mc), lambda i: (i, 0, 0),
                         memory_space=pltpu.SMEM),
            pl.BlockSpec((N_pad, 1, C), lambda i: (0, 0, 0)),
        ],
        out_specs=pl.BlockSpec((1, mc), lambda i: (0, i)),
        scratch_shapes=[
            pltpu.VMEM((mc, C), jnp.float32),
            pltpu.VMEM((mc, C), jnp.float32),
        ],
        compiler_params=pltpu.CompilerParams(
            dimension_semantics=("parallel",),
            vmem_limit_bytes=56 * _MIB),
    )(r_pad, et3, src3, dst3, xn3)

    return score[0, :E]


def kernel(x, R_diagonal, edge_index, edge_type):
    return _distmult(x, R_diagonal, edge_index, edge_type)


# speedup vs baseline: 5.8850x; 1.0442x over previous
import functools

import jax
import jax.numpy as jnp
from jax.experimental import pallas as pl
from jax.experimental.pallas import tpu as pltpu

_MIB = 1024 * 1024


def _round_up(a: int, b: int) -> int:
    return (a + b - 1) // b * b


def _norm_kernel(x_ref, out_ref):
    xb = x_ref[...]
    ss = jnp.sum(xb * xb, axis=1, keepdims=True)
    inv = jax.lax.rsqrt(jnp.maximum(ss, jnp.float32(1e-24)))
    out_ref[...] = xb * inv


def _score_kernel(r_ref, et_ref, sidx_ref, didx_ref, table_ref, out_ref,
                  s_tile, o_tile, *, mc: int, sub: int):
    for c0 in range(0, mc, sub):
        for mi in range(c0, c0 + sub):
            s_tile[mi, :] = table_ref[sidx_ref[0, 0, mi], 0]
        for mi in range(c0, c0 + sub):
            o_tile[mi, :] = table_ref[didx_ref[0, 0, mi], 0]
        chunk = pl.ds(c0, sub)
        t = (s_tile[chunk, :] * o_tile[chunk, :]).astype(jnp.bfloat16)
        q = jax.lax.dot_general(r_ref[...], t, (((1,), (1,)), ((), ())),
                                preferred_element_type=jnp.float32)
        et = et_ref[0, :, chunk]
        krel = jax.lax.broadcasted_iota(jnp.int32, q.shape, 0)
        sel = jnp.where(krel == et, q, jnp.float32(0.0))
        out_ref[0, chunk] = jnp.sum(sel, axis=0)


def _distmult(x, R_diagonal, edge_index, edge_type, *,
              tile_n: int = 2000, mc: int = 4096, sub: int = 512):
    E = int(edge_index.shape[1])
    C = int(x.shape[1])
    N = int(x.shape[0])
    n_rel = int(R_diagonal.shape[0])
    n_rel_pad = _round_up(n_rel, 256)

    tile_n = min(tile_n, _round_up(N, 8))
    N_pad = _round_up(N, tile_n)
    x_p = jnp.pad(x, ((0, N_pad - N), (0, 0))) if N_pad != N else x
    xn = pl.pallas_call(
        _norm_kernel,
        out_shape=jax.ShapeDtypeStruct((N_pad, C), jnp.float32),
        grid=(N_pad // tile_n,),
        in_specs=[pl.BlockSpec((tile_n, C), lambda i: (i, 0))],
        out_specs=pl.BlockSpec((tile_n, C), lambda i: (i, 0)),
        compiler_params=pltpu.CompilerParams(
            dimension_semantics=("parallel",)),
    )(x_p)
    xn3 = xn.reshape(N_pad, 1, C)

    mc = min(mc, _round_up(max(E, 1), 128))
    E_pad = _round_up(max(E, 1), mc)
    src = edge_index[0, :].astype(jnp.int32)
    dst = edge_index[1, :].astype(jnp.int32)
    et = edge_type.astype(jnp.int32)
    if E_pad != E:
        src = jnp.pad(src, (0, E_pad - E))
        dst = jnp.pad(dst, (0, E_pad - E))
        et = jnp.pad(et, (0, E_pad - E))
    n_steps = E_pad // mc
    src3 = src.reshape(n_steps, 1, mc)
    dst3 = dst.reshape(n_steps, 1, mc)
    et3 = et.reshape(n_steps, 1, mc)

    r_pad = jnp.pad(R_diagonal, ((0, n_rel_pad - n_rel), (0, 0))
                    ).astype(jnp.bfloat16)

    score = pl.pallas_call(
        functools.partial(_score_kernel, mc=mc, sub=min(sub, mc)),
        out_shape=jax.ShapeDtypeStruct((1, E_pad), jnp.float32),
        grid=(n_steps,),
        in_specs=[
            pl.BlockSpec((n_rel_pad, C), lambda i: (0, 0)),
            pl.BlockSpec((1, 1, mc), lambda i: (i, 0, 0)),
            pl.BlockSpec((1, 1, mc), lambda i: (i, 0, 0),
                         memory_space=pltpu.SMEM),
            pl.BlockSpec((1, 1, mc), lambda i: (i, 0, 0),
                         memory_space=pltpu.SMEM),
            pl.BlockSpec((N_pad, 1, C), lambda i: (0, 0, 0)),
        ],
        out_specs=pl.BlockSpec((1, mc), lambda i: (0, i)),
        scratch_shapes=[
            pltpu.VMEM((mc, C), jnp.float32),
            pltpu.VMEM((mc, C), jnp.float32),
        ],
        compiler_params=pltpu.CompilerParams(
            dimension_semantics=("parallel",),
            vmem_limit_bytes=56 * _MIB),
    )(r_pad, et3, src3, dst3, xn3)

    return score[0, :E]


def kernel(x, R_diagonal, edge_index, edge_type):
    return _distmult(x, R_diagonal, edge_index, edge_type)


# speedup vs baseline: 7.1561x; 1.2160x over previous
import functools

import jax
import jax.numpy as jnp
from jax.experimental import pallas as pl
from jax.experimental.pallas import tpu as pltpu

_MIB = 1024 * 1024


def _round_up(a: int, b: int) -> int:
    return (a + b - 1) // b * b


def _norm_kernel(x_ref, out_ref):
    xb = x_ref[...]
    ss = jnp.sum(xb * xb, axis=1, keepdims=True)
    inv = jax.lax.rsqrt(jnp.maximum(ss, jnp.float32(1e-24)))
    out_ref[...] = (xb * inv).reshape(out_ref.shape)


def _score_kernel(r_ref, et_ref, sidx_ref, didx_ref, table_ref, out_ref,
                  t_tile, *, mc: int, sub: int):
    for c0 in range(0, mc, sub):
        for mi in range(c0, c0 + sub):
            t_tile[mi, :] = (table_ref[sidx_ref[0, 0, mi], 0] *
                             table_ref[didx_ref[0, 0, mi], 0])
        chunk = pl.ds(c0, sub)
        t = t_tile[chunk, :].astype(jnp.bfloat16)
        q = jax.lax.dot_general(r_ref[...], t, (((1,), (1,)), ((), ())),
                                preferred_element_type=jnp.float32)
        et = et_ref[0, :, chunk]
        krel = jax.lax.broadcasted_iota(jnp.int32, q.shape, 0)
        sel = jnp.where(krel == et, q, jnp.float32(0.0))
        out_ref[0, chunk] = jnp.sum(sel, axis=0)


def _distmult(x, R_diagonal, edge_index, edge_type, *,
              tile_n: int = 2000, mc: int = 4096, sub: int = 512):
    E = int(edge_index.shape[1])
    C = int(x.shape[1])
    N = int(x.shape[0])
    n_rel = int(R_diagonal.shape[0])
    n_rel_pad = _round_up(n_rel, 256)

    tile_n = min(tile_n, _round_up(N, 8))
    N_pad = _round_up(N, tile_n)
    x_p = jnp.pad(x, ((0, N_pad - N), (0, 0))) if N_pad != N else x
    xn3 = pl.pallas_call(
        _norm_kernel,
        out_shape=jax.ShapeDtypeStruct((N_pad, 1, C), jnp.float32),
        grid=(N_pad // tile_n,),
        in_specs=[pl.BlockSpec((tile_n, C), lambda i: (i, 0))],
        out_specs=pl.BlockSpec((tile_n, 1, C), lambda i: (i, 0, 0)),
        compiler_params=pltpu.CompilerParams(
            dimension_semantics=("parallel",)),
    )(x_p)

    mc = min(mc, _round_up(max(E, 1), 128))
    E_pad = _round_up(max(E, 1), mc)
    src = edge_index[0, :].astype(jnp.int32)
    dst = edge_index[1, :].astype(jnp.int32)
    et = edge_type.astype(jnp.int32)
    if E_pad != E:
        src = jnp.pad(src, (0, E_pad - E))
        dst = jnp.pad(dst, (0, E_pad - E))
        et = jnp.pad(et, (0, E_pad - E))
    n_steps = E_pad // mc
    src3 = src.reshape(n_steps, 1, mc)
    dst3 = dst.reshape(n_steps, 1, mc)
    et3 = et.reshape(n_steps, 1, mc)

    r_pad = jnp.pad(R_diagonal, ((0, n_rel_pad - n_rel), (0, 0))
                    ).astype(jnp.bfloat16)

    score = pl.pallas_call(
        functools.partial(_score_kernel, mc=mc, sub=min(sub, mc)),
        out_shape=jax.ShapeDtypeStruct((1, E_pad), jnp.float32),
        grid=(n_steps,),
        in_specs=[
            pl.BlockSpec((n_rel_pad, C), lambda i: (0, 0)),
            pl.BlockSpec((1, 1, mc), lambda i: (i, 0, 0)),
            pl.BlockSpec((1, 1, mc), lambda i: (i, 0, 0),
                         memory_space=pltpu.SMEM),
            pl.BlockSpec((1, 1, mc), lambda i: (i, 0, 0),
                         memory_space=pltpu.SMEM),
            pl.BlockSpec((N_pad, 1, C), lambda i: (0, 0, 0)),
        ],
        out_specs=pl.BlockSpec((1, mc), lambda i: (0, i)),
        scratch_shapes=[
            pltpu.VMEM((mc, C), jnp.float32),
        ],
        compiler_params=pltpu.CompilerParams(
            dimension_semantics=("parallel",),
            vmem_limit_bytes=56 * _MIB),
    )(r_pad, et3, src3, dst3, xn3)

    return score[0, :E]


def kernel(x, R_diagonal, edge_index, edge_type):
    return _distmult(x, R_diagonal, edge_index, edge_type)


# speedup vs baseline: 7.2196x; 1.0089x over previous
import functools

import jax
import jax.numpy as jnp
from jax.experimental import pallas as pl
from jax.experimental.pallas import tpu as pltpu

_MIB = 1024 * 1024


def _round_up(a: int, b: int) -> int:
    return (a + b - 1) // b * b


def _norm_kernel(x_ref, out_ref):
    xb = x_ref[...]
    ss = jnp.sum(xb * xb, axis=1, keepdims=True)
    inv = jax.lax.rsqrt(jnp.maximum(ss, jnp.float32(1e-24)))
    out_ref[...] = (xb * inv).reshape(out_ref.shape)


def _score_kernel(r_ref, et_ref, sidx_ref, didx_ref, table_ref, out_ref,
                  t_tile, *, mc: int, sub: int):
    for c0 in range(0, mc, sub):
        for mi in range(c0, c0 + sub):
            t_tile[mi, :] = (table_ref[sidx_ref[0, 0, mi], 0] *
                             table_ref[didx_ref[0, 0, mi], 0])
        chunk = pl.ds(c0, sub)
        t = t_tile[chunk, :].astype(jnp.bfloat16)
        q = jax.lax.dot_general(r_ref[...], t, (((1,), (1,)), ((), ())),
                                preferred_element_type=jnp.float32)
        et = et_ref[0, :, chunk]
        krel = jax.lax.broadcasted_iota(jnp.int32, q.shape, 0)
        sel = jnp.where(krel == et, q, jnp.float32(0.0))
        out_ref[0, chunk] = jnp.sum(sel, axis=0)


def _distmult(x, R_diagonal, edge_index, edge_type, *,
              tile_n: int = 2000, mc: int = 4096, sub: int = 1024):
    E = int(edge_index.shape[1])
    C = int(x.shape[1])
    N = int(x.shape[0])
    n_rel = int(R_diagonal.shape[0])
    n_rel_pad = _round_up(n_rel, 256)

    tile_n = min(tile_n, _round_up(N, 8))
    N_pad = _round_up(N, tile_n)
    x_p = jnp.pad(x, ((0, N_pad - N), (0, 0))) if N_pad != N else x
    xn3 = pl.pallas_call(
        _norm_kernel,
        out_shape=jax.ShapeDtypeStruct((N_pad, 1, C), jnp.float32),
        grid=(N_pad // tile_n,),
        in_specs=[pl.BlockSpec((tile_n, C), lambda i: (i, 0))],
        out_specs=pl.BlockSpec((tile_n, 1, C), lambda i: (i, 0, 0)),
        compiler_params=pltpu.CompilerParams(
            dimension_semantics=("parallel",)),
    )(x_p)

    mc = min(mc, _round_up(max(E, 1), 128))
    E_pad = _round_up(max(E, 1), mc)
    src = edge_index[0, :].astype(jnp.int32)
    dst = edge_index[1, :].astype(jnp.int32)
    et = edge_type.astype(jnp.int32)
    if E_pad != E:
        src = jnp.pad(src, (0, E_pad - E))
        dst = jnp.pad(dst, (0, E_pad - E))
        et = jnp.pad(et, (0, E_pad - E))
    n_steps = E_pad // mc
    src3 = src.reshape(n_steps, 1, mc)
    dst3 = dst.reshape(n_steps, 1, mc)
    et3 = et.reshape(n_steps, 1, mc)

    r_pad = jnp.pad(R_diagonal, ((0, n_rel_pad - n_rel), (0, 0))
                    ).astype(jnp.bfloat16)

    score = pl.pallas_call(
        functools.partial(_score_kernel, mc=mc, sub=min(sub, mc)),
        out_shape=jax.ShapeDtypeStruct((1, E_pad), jnp.float32),
        grid=(n_steps,),
        in_specs=[
            pl.BlockSpec((n_rel_pad, C), lambda i: (0, 0)),
            pl.BlockSpec((1, 1, mc), lambda i: (i, 0, 0)),
            pl.BlockSpec((1, 1, mc), lambda i: (i, 0, 0),
                         memory_space=pltpu.SMEM),
            pl.BlockSpec((1, 1, mc), lambda i: (i, 0, 0),
                         memory_space=pltpu.SMEM),
            pl.BlockSpec((N_pad, 1, C), lambda i: (0, 0, 0)),
        ],
        out_specs=pl.BlockSpec((1, mc), lambda i: (0, i)),
        scratch_shapes=[
            pltpu.VMEM((mc, C), jnp.float32),
        ],
        compiler_params=pltpu.CompilerParams(
            dimension_semantics=("parallel",),
            vmem_limit_bytes=56 * _MIB),
    )(r_pad, et3, src3, dst3, xn3)

    return score[0, :E]


def kernel(x, R_diagonal, edge_index, edge_type):
    return _distmult(x, R_diagonal, edge_index, edge_type)


# speedup vs baseline: 7.2685x; 1.0068x over previous
import functools
import math

import jax
import jax.numpy as jnp
from jax.experimental import pallas as pl
from jax.experimental.pallas import tpu as pltpu

_MIB = 1024 * 1024


def _round_up(a: int, b: int) -> int:
    return (a + b - 1) // b * b


def _norm_kernel(x_ref, out_ref):
    xb = x_ref[...]
    ss = jnp.sum(xb * xb, axis=1, keepdims=True)
    inv = jax.lax.rsqrt(jnp.maximum(ss, jnp.float32(1e-24)))
    out_ref[...] = (xb * inv).reshape(out_ref.shape)


def _score_kernel(r_ref, et_ref, sidx_ref, didx_ref, table_ref, out_ref,
                  t_tile, *, mc: int, sub: int):
    for c0 in range(0, mc, sub):
        for mi in range(c0, c0 + sub):
            t_tile[mi, :] = (table_ref[sidx_ref[0, 0, mi], 0] *
                             table_ref[didx_ref[0, 0, mi], 0])
        chunk = pl.ds(c0, sub)
        t = t_tile[chunk, :].astype(jnp.bfloat16)
        q = jax.lax.dot_general(r_ref[...], t, (((1,), (1,)), ((), ())),
                                preferred_element_type=jnp.float32)
        et = et_ref[0, :, chunk]
        krel = jax.lax.broadcasted_iota(jnp.int32, q.shape, 0)
        sel = jnp.where(krel == et, q, jnp.float32(0.0))
        out_ref[0, chunk] = jnp.sum(sel, axis=0)


def _distmult(x, R_diagonal, edge_index, edge_type, *,
              tile_n: int = 2000, mc: int = 4096, sub: int = 1024):
    E = int(edge_index.shape[1])
    C = int(x.shape[1])
    N = int(x.shape[0])
    n_rel = int(R_diagonal.shape[0])
    n_rel_pad = _round_up(n_rel, 256)

    tile_n = min(tile_n, _round_up(N, 8))
    N_pad = _round_up(N, tile_n)
    x_p = jnp.pad(x, ((0, N_pad - N), (0, 0))) if N_pad != N else x
    xn3 = pl.pallas_call(
        _norm_kernel,
        out_shape=jax.ShapeDtypeStruct((N_pad, 1, C), jnp.float32),
        grid=(N_pad // tile_n,),
        in_specs=[pl.BlockSpec((tile_n, C), lambda i: (i, 0))],
        out_specs=pl.BlockSpec((tile_n, 1, C), lambda i: (i, 0, 0)),
        compiler_params=pltpu.CompilerParams(
            dimension_semantics=("parallel",)),
    )(x_p)

    mc = min(mc, _round_up(max(E, 1), 128))
    E_pad = _round_up(max(E, 1), mc)
    src = edge_index[0, :].astype(jnp.int32)
    dst = edge_index[1, :].astype(jnp.int32)
    et = edge_type.astype(jnp.int32)
    if E_pad != E:
        src = jnp.pad(src, (0, E_pad - E))
        dst = jnp.pad(dst, (0, E_pad - E))
        et = jnp.pad(et, (0, E_pad - E))
    n_steps = E_pad // mc
    src3 = src.reshape(n_steps, 1, mc)
    dst3 = dst.reshape(n_steps, 1, mc)
    et3 = et.reshape(n_steps, 1, mc)

    r_pad = jnp.pad(R_diagonal, ((0, n_rel_pad - n_rel), (0, 0))
                    ).astype(jnp.bfloat16)

    score = pl.pallas_call(
        functools.partial(_score_kernel, mc=mc, sub=math.gcd(min(sub, mc), mc)),
        out_shape=jax.ShapeDtypeStruct((1, E_pad), jnp.float32),
        grid=(n_steps,),
        in_specs=[
            pl.BlockSpec((n_rel_pad, C), lambda i: (0, 0)),
            pl.BlockSpec((1, 1, mc), lambda i: (i, 0, 0)),
            pl.BlockSpec((1, 1, mc), lambda i: (i, 0, 0),
                         memory_space=pltpu.SMEM),
            pl.BlockSpec((1, 1, mc), lambda i: (i, 0, 0),
                         memory_space=pltpu.SMEM),
            pl.BlockSpec((N_pad, 1, C), lambda i: (0, 0, 0)),
        ],
        out_specs=pl.BlockSpec((1, mc), lambda i: (0, i)),
        scratch_shapes=[
            pltpu.VMEM((mc, C), jnp.float32),
        ],
        compiler_params=pltpu.CompilerParams(
            dimension_semantics=("parallel",),
            vmem_limit_bytes=56 * _MIB),
    )(r_pad, et3, src3, dst3, xn3)

    return score[0, :E]


def kernel(x, R_diagonal, edge_index, edge_type):
    return _distmult(x, R_diagonal, edge_index, edge_type)
